# baseline (speedup 1.0000x reference)
import jax
import jax.numpy as jnp
import numpy as np
from jax.experimental import pallas as pl

jax.config.update("jax_enable_x64", True)

_I0 = np.int32(0)


def _ord_kernel(x_ref, ord_ref, dec_ref):
    i = pl.program_id(1)
    lo = jnp.asarray(1e-8, jnp.float32)
    hi = jnp.asarray(10000.0, jnp.float32)
    a = jnp.clip(x_ref[0, 0, :, :], lo, hi)
    b = jnp.clip(x_ref[0, 1, :, :], lo, hi)
    d = b - a
    ord_ref[0, 0, :, :] = jax.nn.sigmoid(d)
    cnt = (d > 0).astype(jnp.int32)

    @pl.when(i == 0)
    def _init():
        dec_ref[0, 0, :, :] = cnt

    @pl.when(i != 0)
    def _acc():
        dec_ref[0, 0, :, :] += cnt


def _half_call(x, ofs, n_pairs):
    N, C, H, W = x.shape
    o = np.int32(ofs)
    return pl.pallas_call(
        _ord_kernel,
        grid=(N, n_pairs),
        in_specs=[pl.BlockSpec((1, 2, H, W), lambda n, i: (n, i + o, _I0, _I0))],
        out_specs=[
            pl.BlockSpec((1, 1, H, W), lambda n, i: (n, i, _I0, _I0)),
            pl.BlockSpec((1, 1, H, W), lambda n, i: (n, _I0, _I0, _I0)),
        ],
        out_shape=[
            jax.ShapeDtypeStruct((N, n_pairs, H, W), jnp.float32),
            jax.ShapeDtypeStruct((N, 1, H, W), jnp.int32),
        ],
    )(x)


def kernel(x):
    N, C, H, W = x.shape
    ord_num = C // 2
    half = ord_num // 2
    ord_a, dec_a = _half_call(x, 0, half)
    ord_b, dec_b = _half_call(x, half, ord_num - half)
    ord64 = jnp.concatenate(
        [ord_a.astype(jnp.float64), ord_b.astype(jnp.float64)], axis=1
    )
    return ((dec_a + dec_b).astype(jnp.int64), ord64)


# speedup vs baseline: 1.0350x; 1.0350x over previous
import jax
import jax.numpy as jnp
import numpy as np
from jax.experimental import pallas as pl

jax.config.update("jax_enable_x64", True)

_I0 = np.int32(0)


def _ord_kernel(x_ref, ord_ref, dec_ref):
    i = pl.program_id(1)
    lo = jnp.asarray(1e-8, jnp.float32)
    hi = jnp.asarray(10000.0, jnp.float32)
    a = jnp.clip(x_ref[0, 0, :, :], lo, hi)
    b = jnp.clip(x_ref[0, 1, :, :], lo, hi)
    d = b - a
    ord_ref[0, 0, :, :] = jax.nn.sigmoid(d)
    cnt = (d > 0).astype(jnp.int32)

    @pl.when(i == 0)
    def _init():
        dec_ref[0, 0, :, :] = cnt

    @pl.when(i != 0)
    def _acc():
        dec_ref[0, 0, :, :] += cnt


def kernel(x):
    N, C, H, W = x.shape
    ord_num = C // 2
    ord32, dec32 = pl.pallas_call(
        _ord_kernel,
        grid=(N, ord_num),
        in_specs=[pl.BlockSpec((1, 2, H, W), lambda n, i: (n, i, _I0, _I0))],
        out_specs=[
            pl.BlockSpec((1, 1, H, W), lambda n, i: (n, i, _I0, _I0)),
            pl.BlockSpec((1, 1, H, W), lambda n, i: (n, _I0, _I0, _I0)),
        ],
        out_shape=[
            jax.ShapeDtypeStruct((N, ord_num, H, W), jnp.float32),
            jax.ShapeDtypeStruct((N, 1, H, W), jnp.int32),
        ],
    )(x)
    return (dec32.astype(jnp.int64), ord32.astype(jnp.float64))
